# speedup vs baseline: 2.3651x; 2.3651x over previous
import functools

import jax
import jax.numpy as jnp
from jax import lax
from jax.experimental import pallas as pl
from jax.experimental.pallas import tpu as pltpu
from jax.experimental.pallas import tpu_sc as plsc

NC = 2
NS = 16
NW = NC * NS
C = 32


def _make_sc_gather(n, d, dtype):
    b_per_w = n // NW
    n_chunks = b_per_w // C
    mesh = plsc.VectorSubcoreMesh(core_axis_name="c", subcore_axis_name="s")

    @functools.partial(
        pl.kernel,
        out_type=jax.ShapeDtypeStruct((n, d), dtype),
        mesh=mesh,
        scratch_types=[
            pltpu.VMEM((n_chunks, C), jnp.int32),
            pltpu.VMEM((C, d), dtype),
            pltpu.VMEM((C, d), dtype),
            pltpu.SemaphoreType.DMA,
            pltpu.SemaphoreType.DMA,
        ],
    )
    def gather_kernel(idx_hbm, table_hbm, out_hbm, idx_v, buf0, buf1, sem0, sem1):
        wid = lax.axis_index("s") * NC + lax.axis_index("c")
        base = wid * b_per_w
        pltpu.sync_copy(idx_hbm.at[wid], idx_v)

        pltpu.async_copy(table_hbm.at[idx_v.at[0]], buf0, sem0)

        def body(j, _):
            @pl.when(j + 1 < n_chunks)
            def _():
                pltpu.async_copy(table_hbm.at[idx_v.at[j + 1]], buf1, sem1)

            pltpu.make_async_copy(table_hbm.at[idx_v.at[0]], buf0, sem0).wait()
            pltpu.sync_copy(buf0, out_hbm.at[pl.ds(base + j * C, C)])

            @pl.when(j + 2 < n_chunks)
            def _():
                pltpu.async_copy(table_hbm.at[idx_v.at[j + 2]], buf0, sem0)

            @pl.when(j + 1 < n_chunks)
            def _():
                pltpu.make_async_copy(
                    table_hbm.at[idx_v.at[0]], buf1, sem1
                ).wait()
                pltpu.sync_copy(buf1, out_hbm.at[pl.ds(base + (j + 1) * C, C)])

            return ()

        lax.fori_loop(0, n_chunks // 2, lambda i, c: body(i * 2, c), (), unroll=False)

    return gather_kernel


def kernel(x, table):
    b, s = x.shape
    v, d = table.shape
    n = b * s
    idx = x.reshape(NW, (n // NW) // C, C).astype(jnp.int32)
    out = _make_sc_gather(n, d, table.dtype)(idx, table)
    return out.reshape(b, s, d)
